# baseline (speedup 1.0000x reference)
import functools

import jax
import jax.numpy as jnp
from jax.experimental import pallas as pl
from jax.experimental.pallas import tpu as pltpu

_BATCH = 64
_TOPK = 512
_DIM = 1024
_HID = 2048
_MEM = 128


def _fused_kernel(w_ref, e_ref, wt_ref, b_ref, g_ref, bt_ref, out_ref):
    w_row = jnp.maximum(w_ref[0], 0.0)
    s = jnp.maximum(jnp.sum(w_row), 1e-6)
    wn_row = w_row / s

    wn_col = jnp.transpose(wn_row, (1, 0))
    wi = jnp.broadcast_to(wn_col, (_TOPK, _TOPK))
    wj = jnp.broadcast_to(wn_row, (_TOPK, _TOPK))
    ii = jax.lax.broadcasted_iota(jnp.int32, (_TOPK, _TOPK), 0)
    jj = jax.lax.broadcasted_iota(jnp.int32, (_TOPK, _TOPK), 1)
    beats = (wi > wj) | ((wi == wj) & (ii < jj))
    rank = jnp.sum(beats.astype(jnp.int32), axis=0, keepdims=True)

    t_iota = jax.lax.broadcasted_iota(jnp.int32, (_MEM, _TOPK), 0)
    p_mat = (jnp.broadcast_to(rank, (_MEM, _TOPK)) == t_iota).astype(jnp.float32)

    sw = jnp.sum(p_mat * jnp.broadcast_to(wn_row, (_MEM, _TOPK)),
                 axis=1, keepdims=True)

    e_clip = jnp.clip(e_ref[0], -5.0, 5.0)
    sel = jax.lax.dot_general(
        p_mat, e_clip, (((1,), (0,)), ((), ())),
        preferred_element_type=jnp.float32,
        precision=jax.lax.Precision.HIGHEST)

    tokens = jax.lax.dot_general(
        sel, wt_ref[...], (((1,), (1,)), ((), ())),
        preferred_element_type=jnp.float32,
        precision=jax.lax.Precision.HIGHEST)
    tokens = (tokens + b_ref[...]) * sw
    tokens = jnp.clip(tokens, -5.0, 5.0)

    mean = jnp.mean(tokens, axis=-1, keepdims=True)
    cent = tokens - mean
    var = jnp.mean(cent * cent, axis=-1, keepdims=True)
    out = cent * jax.lax.rsqrt(var + 1e-5) * g_ref[...] + bt_ref[...]
    out_ref[0] = out


@jax.jit
def kernel(image_embeds, weights, W, b, gamma, beta):
    b2 = b.reshape(1, _HID)
    g2 = gamma.reshape(1, _HID)
    bt2 = beta.reshape(1, _HID)
    return pl.pallas_call(
        _fused_kernel,
        grid=(_BATCH,),
        in_specs=[
            pl.BlockSpec((1, 1, _TOPK), lambda i: (i, 0, 0)),
            pl.BlockSpec((1, _TOPK, _DIM), lambda i: (i, 0, 0)),
            pl.BlockSpec((_HID, _DIM), lambda i: (0, 0)),
            pl.BlockSpec((1, _HID), lambda i: (0, 0)),
            pl.BlockSpec((1, _HID), lambda i: (0, 0)),
            pl.BlockSpec((1, _HID), lambda i: (0, 0)),
        ],
        out_specs=pl.BlockSpec((1, _MEM, _HID), lambda i: (i, 0, 0)),
        out_shape=jax.ShapeDtypeStruct((_BATCH, _MEM, _HID), jnp.float32),
        compiler_params=pltpu.CompilerParams(
            dimension_semantics=("arbitrary",),
        ),
    )(weights.reshape(_BATCH, 1, _TOPK), image_embeds, W, b2, g2, bt2)


# speedup vs baseline: 3.9806x; 3.9806x over previous
import functools

import jax
import jax.numpy as jnp
from jax.experimental import pallas as pl
from jax.experimental.pallas import tpu as pltpu

_BATCH = 64
_TOPK = 512
_DIM = 1024
_HID = 2048
_MEM = 128


def _fused_kernel(w_ref, e_ref, wt_ref, b_ref, g_ref, bt_ref, out_ref):
    w_row = jnp.maximum(w_ref[0], 0.0)
    s = jnp.maximum(jnp.sum(w_row), 1e-6)
    wn_row = w_row / s

    wn_col = jnp.transpose(wn_row, (1, 0))
    wi = jnp.broadcast_to(wn_col, (_TOPK, _TOPK))
    wj = jnp.broadcast_to(wn_row, (_TOPK, _TOPK))
    ii = jax.lax.broadcasted_iota(jnp.int32, (_TOPK, _TOPK), 0)
    jj = jax.lax.broadcasted_iota(jnp.int32, (_TOPK, _TOPK), 1)
    beats = (wi > wj) | ((wi == wj) & (ii < jj))
    rank = jnp.sum(beats.astype(jnp.int32), axis=0, keepdims=True)

    t_iota = jax.lax.broadcasted_iota(jnp.int32, (_MEM, _TOPK), 0)
    p_mat = (jnp.broadcast_to(rank, (_MEM, _TOPK)) == t_iota).astype(jnp.float32)

    sw = jnp.sum(p_mat * jnp.broadcast_to(wn_row, (_MEM, _TOPK)),
                 axis=1, keepdims=True)

    e_clip = jnp.clip(e_ref[0], -5.0, 5.0)
    sel = jax.lax.dot_general(
        p_mat, e_clip, (((1,), (0,)), ((), ())),
        preferred_element_type=jnp.float32,
        precision=jax.lax.Precision.DEFAULT)

    tokens = jax.lax.dot_general(
        sel, wt_ref[...], (((1,), (1,)), ((), ())),
        preferred_element_type=jnp.float32,
        precision=jax.lax.Precision.DEFAULT)
    tokens = (tokens + b_ref[...]) * sw
    tokens = jnp.clip(tokens, -5.0, 5.0)

    mean = jnp.mean(tokens, axis=-1, keepdims=True)
    cent = tokens - mean
    var = jnp.mean(cent * cent, axis=-1, keepdims=True)
    out = cent * jax.lax.rsqrt(var + 1e-5) * g_ref[...] + bt_ref[...]
    out_ref[0] = out


@jax.jit
def kernel(image_embeds, weights, W, b, gamma, beta):
    b2 = b.reshape(1, _HID)
    g2 = gamma.reshape(1, _HID)
    bt2 = beta.reshape(1, _HID)
    return pl.pallas_call(
        _fused_kernel,
        grid=(_BATCH,),
        in_specs=[
            pl.BlockSpec((1, 1, _TOPK), lambda i: (i, 0, 0)),
            pl.BlockSpec((1, _TOPK, _DIM), lambda i: (i, 0, 0)),
            pl.BlockSpec((_HID, _DIM), lambda i: (0, 0)),
            pl.BlockSpec((1, _HID), lambda i: (0, 0)),
            pl.BlockSpec((1, _HID), lambda i: (0, 0)),
            pl.BlockSpec((1, _HID), lambda i: (0, 0)),
        ],
        out_specs=pl.BlockSpec((1, _MEM, _HID), lambda i: (i, 0, 0)),
        out_shape=jax.ShapeDtypeStruct((_BATCH, _MEM, _HID), jnp.float32),
        compiler_params=pltpu.CompilerParams(
            dimension_semantics=("arbitrary",),
        ),
    )(weights.reshape(_BATCH, 1, _TOPK), image_embeds, W, b2, g2, bt2)


# speedup vs baseline: 6.6832x; 1.6789x over previous
import functools

import jax
import jax.numpy as jnp
from jax.experimental import pallas as pl
from jax.experimental.pallas import tpu as pltpu

_BATCH = 64
_TOPK = 512
_DIM = 1024
_HID = 2048
_MEM = 128
_BB = 2


def _fused_kernel(w_ref, e_ref, wt_ref, b_ref, g_ref, bt_ref, out_ref):
    w = jnp.maximum(w_ref[...], 0.0)
    s = jnp.maximum(jnp.sum(w, axis=2, keepdims=True), 1e-6)
    wn = w / s

    wn_col = jnp.transpose(wn, (0, 2, 1))
    wi = jnp.broadcast_to(wn_col, (_BB, _TOPK, _TOPK))
    wj = jnp.broadcast_to(wn, (_BB, _TOPK, _TOPK))
    ii = jax.lax.broadcasted_iota(jnp.int32, (_BB, _TOPK, _TOPK), 1)
    jj = jax.lax.broadcasted_iota(jnp.int32, (_BB, _TOPK, _TOPK), 2)
    beats = (wi > wj) | ((wi == wj) & (ii < jj))
    rank = jnp.sum(beats.astype(jnp.int32), axis=1, keepdims=True)

    t_iota = jax.lax.broadcasted_iota(jnp.int32, (_BB, _MEM, _TOPK), 1)
    p_mat = (jnp.broadcast_to(rank, (_BB, _MEM, _TOPK)) == t_iota
             ).astype(jnp.float32)

    sw = jnp.sum(p_mat * jnp.broadcast_to(wn, (_BB, _MEM, _TOPK)),
                 axis=2, keepdims=True)

    sel = jax.lax.dot_general(
        p_mat, e_ref[...], (((2,), (1,)), ((0,), (0,))),
        preferred_element_type=jnp.float32,
        precision=jax.lax.Precision.DEFAULT)
    sel = jnp.clip(sel.reshape(_BB * _MEM, _DIM), -5.0, 5.0)

    tokens = jax.lax.dot_general(
        sel, wt_ref[...], (((1,), (1,)), ((), ())),
        preferred_element_type=jnp.float32,
        precision=jax.lax.Precision.DEFAULT)
    tokens = (tokens + b_ref[...]) * sw.reshape(_BB * _MEM, 1)
    tokens = jnp.clip(tokens, -5.0, 5.0)

    mean = jnp.mean(tokens, axis=-1, keepdims=True)
    cent = tokens - mean
    var = jnp.mean(cent * cent, axis=-1, keepdims=True)
    out = cent * jax.lax.rsqrt(var + 1e-5) * g_ref[...] + bt_ref[...]
    out_ref[...] = out.reshape(_BB, _MEM, _HID)


@jax.jit
def kernel(image_embeds, weights, W, b, gamma, beta):
    b2 = b.reshape(1, _HID)
    g2 = gamma.reshape(1, _HID)
    bt2 = beta.reshape(1, _HID)
    return pl.pallas_call(
        _fused_kernel,
        grid=(_BATCH // _BB,),
        in_specs=[
            pl.BlockSpec((_BB, 1, _TOPK), lambda i: (i, 0, 0)),
            pl.BlockSpec((_BB, _TOPK, _DIM), lambda i: (i, 0, 0)),
            pl.BlockSpec((_HID, _DIM), lambda i: (0, 0)),
            pl.BlockSpec((1, _HID), lambda i: (0, 0)),
            pl.BlockSpec((1, _HID), lambda i: (0, 0)),
            pl.BlockSpec((1, _HID), lambda i: (0, 0)),
        ],
        out_specs=pl.BlockSpec((_BB, _MEM, _HID), lambda i: (i, 0, 0)),
        out_shape=jax.ShapeDtypeStruct((_BATCH, _MEM, _HID), jnp.float32),
        compiler_params=pltpu.CompilerParams(
            dimension_semantics=("arbitrary",),
        ),
    )(weights.reshape(_BATCH, 1, _TOPK), image_embeds, W, b2, g2, bt2)


# speedup vs baseline: 7.6002x; 1.1372x over previous
import functools

import jax
import jax.numpy as jnp
from jax.experimental import pallas as pl
from jax.experimental.pallas import tpu as pltpu

_BATCH = 64
_TOPK = 512
_DIM = 1024
_HID = 2048
_MEM = 128
_BB = 4


def _fused_kernel(w_ref, e_ref, wt_ref, b_ref, g_ref, bt_ref, out_ref):
    w = jnp.maximum(w_ref[...], 0.0)
    s = jnp.maximum(jnp.sum(w, axis=2, keepdims=True), 1e-6)
    wn = w / s

    wn_col = jnp.transpose(wn, (0, 2, 1))
    wi = jnp.broadcast_to(wn_col, (_BB, _TOPK, _TOPK))
    wj = jnp.broadcast_to(wn, (_BB, _TOPK, _TOPK))
    ii = jax.lax.broadcasted_iota(jnp.int32, (_BB, _TOPK, _TOPK), 1)
    jj = jax.lax.broadcasted_iota(jnp.int32, (_BB, _TOPK, _TOPK), 2)
    beats = (wi > wj) | ((wi == wj) & (ii < jj))
    rank = jnp.sum(beats.astype(jnp.int32), axis=1, keepdims=True)

    t_iota = jax.lax.broadcasted_iota(jnp.int32, (_BB, _MEM, _TOPK), 1)
    p_mat = (jnp.broadcast_to(rank, (_BB, _MEM, _TOPK)) == t_iota
             ).astype(jnp.float32)

    sw = jnp.sum(p_mat * jnp.broadcast_to(wn, (_BB, _MEM, _TOPK)),
                 axis=2, keepdims=True)

    sel = jax.lax.dot_general(
        p_mat, e_ref[...], (((2,), (1,)), ((0,), (0,))),
        preferred_element_type=jnp.float32,
        precision=jax.lax.Precision.DEFAULT)
    sel = jnp.clip(sel.reshape(_BB * _MEM, _DIM), -5.0, 5.0)

    tokens = jax.lax.dot_general(
        sel, wt_ref[...], (((1,), (1,)), ((), ())),
        preferred_element_type=jnp.float32,
        precision=jax.lax.Precision.DEFAULT)
    tokens = (tokens + b_ref[...]) * sw.reshape(_BB * _MEM, 1)
    tokens = jnp.clip(tokens, -5.0, 5.0)

    mean = jnp.mean(tokens, axis=-1, keepdims=True)
    cent = tokens - mean
    var = jnp.mean(cent * cent, axis=-1, keepdims=True)
    out = cent * jax.lax.rsqrt(var + 1e-5) * g_ref[...] + bt_ref[...]
    out_ref[...] = out.reshape(_BB, _MEM, _HID)


@jax.jit
def kernel(image_embeds, weights, W, b, gamma, beta):
    b2 = b.reshape(1, _HID)
    g2 = gamma.reshape(1, _HID)
    bt2 = beta.reshape(1, _HID)
    return pl.pallas_call(
        _fused_kernel,
        grid=(_BATCH // _BB,),
        in_specs=[
            pl.BlockSpec((_BB, 1, _TOPK), lambda i: (i, 0, 0)),
            pl.BlockSpec((_BB, _TOPK, _DIM), lambda i: (i, 0, 0)),
            pl.BlockSpec((_HID, _DIM), lambda i: (0, 0)),
            pl.BlockSpec((1, _HID), lambda i: (0, 0)),
            pl.BlockSpec((1, _HID), lambda i: (0, 0)),
            pl.BlockSpec((1, _HID), lambda i: (0, 0)),
        ],
        out_specs=pl.BlockSpec((_BB, _MEM, _HID), lambda i: (i, 0, 0)),
        out_shape=jax.ShapeDtypeStruct((_BATCH, _MEM, _HID), jnp.float32),
        compiler_params=pltpu.CompilerParams(
            dimension_semantics=("arbitrary",),
        ),
    )(weights.reshape(_BATCH, 1, _TOPK), image_embeds, W, b2, g2, bt2)
